# baseline (speedup 1.0000x reference)
import functools

import jax
import jax.numpy as jnp
from jax import lax
from jax.experimental import pallas as pl
from jax.experimental.pallas import tpu as pltpu
from jax.experimental.pallas import tpu_sc as plsc

D = 128
D2 = 2 * D
CHUNK = 256
HCHUNK = CHUNK // 2
NC = 2
NS = 16
NW = NC * NS
SCHUNKS = 20
NBUF = 2
L = 16


@functools.lru_cache(maxsize=None)
def _make_sc_embed(n_idx: int):
    idx_per_w = n_idx // NW
    chunks_per_w = idx_per_w // CHUNK
    n_stages = chunks_per_w // SCHUNKS
    groups_per_stage = SCHUNKS // NBUF
    assert n_idx == idx_per_w * NW
    assert idx_per_w == chunks_per_w * CHUNK
    assert chunks_per_w == n_stages * SCHUNKS and n_stages % 2 == 0
    assert SCHUNKS == groups_per_stage * NBUF

    mesh = plsc.VectorSubcoreMesh(core_axis_name="c", subcore_axis_name="s")

    @functools.partial(
        pl.kernel,
        mesh=mesh,
        out_type=jax.ShapeDtypeStruct((n_idx // 2, 2, D), jnp.float32),
        scratch_types=(
            [pltpu.VMEM((SCHUNKS * CHUNK,), jnp.int32) for _ in range(2)]
            + [pltpu.VMEM_SHARED((25, 2, D), jnp.float32)]
            + [pltpu.VMEM((HCHUNK, 2, D), jnp.float32) for _ in range(NBUF)]
            + [pltpu.VMEM((1, HCHUNK), jnp.int32) for _ in range(NBUF)]
            + [pltpu.SemaphoreType.DMA for _ in range(2 * NBUF + 2)]
        ),
    )
    def sc_embed(x_hbm, w2_hbm, out_hbm, idx0, idx1, w_v, *bufs_and_sems):
        idx_bufs = (idx0, idx1)
        rows = bufs_and_sems[:NBUF]
        pidx = bufs_and_sems[NBUF : 2 * NBUF]
        gsems = bufs_and_sems[2 * NBUF : 3 * NBUF]
        wsems = bufs_and_sems[3 * NBUF : 4 * NBUF]
        isems = bufs_and_sems[4 * NBUF : 4 * NBUF + 2]
        cid = lax.axis_index("c")
        sid = lax.axis_index("s")
        wid = sid * NC + cid
        idx0_w = wid * idx_per_w
        hrow0_w = wid * (idx_per_w // 2)
        lanes = lax.iota(jnp.int32, L)

        @pl.when(sid == 0)
        def _stage_table():
            pltpu.sync_copy(w2_hbm, w_v)

        plsc.subcore_barrier()

        def idx_fetch(st, slot):
            pltpu.make_async_copy(
                x_hbm.at[pl.ds(idx0_w + st * SCHUNKS * CHUNK, SCHUNKS * CHUNK)],
                idx_bufs[slot],
                isems[slot],
            ).start()

        def process_stage(st, slot):
            idx_v = idx_bufs[slot]
            pltpu.make_async_copy(
                x_hbm.at[pl.ds(idx0_w, SCHUNKS * CHUNK)], idx_v, isems[slot]
            ).wait()

            def group_body(g, carry):
                gchunk = st * SCHUNKS + g * NBUF
                gathers = []
                for b in range(NBUF):
                    @pl.when(gchunk + b >= NBUF)
                    def _wait_prev_write():
                        pltpu.make_async_copy(
                            rows[b], out_hbm.at[pl.ds(0, HCHUNK)], wsems[b]
                        ).wait()

                    cbase = (g * NBUF + b) * CHUNK
                    for t in range(CHUNK // (2 * L)):
                        va = idx_v[pl.ds(cbase + 2 * L * t, L)]
                        vb = idx_v[pl.ds(cbase + 2 * L * t + L, L)]
                        ia = (2 * lanes) & (L - 1)
                        ga = jnp.take_along_axis(va, ia, axis=0)
                        gb = jnp.take_along_axis(vb, ia, axis=0)
                        xe = jnp.where(lanes < L // 2, ga, gb)
                        ga1 = jnp.take_along_axis(va, ia + 1, axis=0)
                        gb1 = jnp.take_along_axis(vb, ia + 1, axis=0)
                        xo = jnp.where(lanes < L // 2, ga1, gb1)
                        pidx[b][0, pl.ds(L * t, L)] = xe * 5 + xo

                    cp = pltpu.make_async_copy(
                        w_v.at[pidx[b].at[0]], rows[b], gsems[b]
                    )
                    cp.start()
                    gathers.append(cp)
                for b in range(NBUF):
                    gathers[b].wait()
                    pltpu.make_async_copy(
                        rows[b],
                        out_hbm.at[pl.ds(hrow0_w + (gchunk + b) * HCHUNK, HCHUNK)],
                        wsems[b],
                    ).start()
                return carry

            lax.fori_loop(0, groups_per_stage, group_body, 0)
            @pl.when(st + 2 < n_stages)
            def _prefetch_next():
                idx_fetch(st + 2, slot)

        idx_fetch(0, 0)
        if n_stages > 1:
            idx_fetch(1, 1)

        def super_body(ss, carry):
            process_stage(2 * ss, 0)
            process_stage(2 * ss + 1, 1)
            return carry

        lax.fori_loop(0, n_stages // 2, super_body, 0)
        for b in range(NBUF):
            pltpu.make_async_copy(
                rows[b], out_hbm.at[pl.ds(0, HCHUNK)], wsems[b]
            ).wait()

    return sc_embed


def kernel(x, weight):
    n, m = x.shape
    total = n * m
    x1 = x.astype(jnp.int32).reshape(total)
    w2 = jnp.concatenate(
        [jnp.repeat(weight, 5, axis=0), jnp.tile(weight, (5, 1))], axis=1
    ).reshape(25, 2, D)
    out = _make_sc_embed(total)(x1, w2)
    return out.reshape(n, m, D)


# speedup vs baseline: 1.2675x; 1.2675x over previous
import functools

import jax
import jax.numpy as jnp
from jax import lax
from jax.experimental import pallas as pl
from jax.experimental.pallas import tpu as pltpu
from jax.experimental.pallas import tpu_sc as plsc

D = 128
CHUNK = 128
NC = 2
NS = 16
NW = NC * NS
IDX_STAGE = 40
NBUF = 4


@functools.lru_cache(maxsize=None)
def _make_sc_embed(n_rows: int):
    rows_per_w = n_rows // NW
    n_stages = rows_per_w // IDX_STAGE
    groups_per_stage = IDX_STAGE // NBUF
    assert n_rows == rows_per_w * NW
    assert rows_per_w == n_stages * IDX_STAGE and n_stages % 2 == 0
    assert IDX_STAGE == groups_per_stage * NBUF

    mesh = plsc.VectorSubcoreMesh(core_axis_name="c", subcore_axis_name="s")

    @functools.partial(
        pl.kernel,
        mesh=mesh,
        out_type=jax.ShapeDtypeStruct((n_rows * CHUNK, D), jnp.float32),
        scratch_types=(
            [pltpu.VMEM((IDX_STAGE, CHUNK), jnp.int32) for _ in range(2)]
            + [pltpu.VMEM_SHARED((5, D), jnp.float32)]
            + [pltpu.VMEM((CHUNK, D), jnp.float32) for _ in range(NBUF)]
            + [pltpu.SemaphoreType.DMA for _ in range(2 * NBUF + 2)]
        ),
    )
    def sc_embed(x_hbm, w_hbm, out_hbm, idx0, idx1, w_v, *bufs_and_sems):
        idx_bufs = (idx0, idx1)
        rows = bufs_and_sems[:NBUF]
        gsems = bufs_and_sems[NBUF : 2 * NBUF]
        wsems = bufs_and_sems[2 * NBUF : 3 * NBUF]
        isems = bufs_and_sems[3 * NBUF : 3 * NBUF + 2]
        cid = lax.axis_index("c")
        sid = lax.axis_index("s")
        wid = sid * NC + cid
        row0 = wid * rows_per_w

        @pl.when(sid == 0)
        def _stage_table():
            pltpu.sync_copy(w_hbm, w_v)

        plsc.subcore_barrier()

        def idx_fetch(st, slot):
            pltpu.make_async_copy(
                x_hbm.at[pl.ds(row0 + st * IDX_STAGE, IDX_STAGE)],
                idx_bufs[slot],
                isems[slot],
            ).start()

        def process_stage(st, slot):
            idx_v = idx_bufs[slot]
            pltpu.make_async_copy(
                x_hbm.at[pl.ds(row0, IDX_STAGE)], idx_v, isems[slot]
            ).wait()
            srow = row0 + st * IDX_STAGE

            def group_body(g, carry):
                grow = srow + g * NBUF
                chunk0 = st * IDX_STAGE + g * NBUF
                gathers = []
                for b in range(NBUF):
                    @pl.when(chunk0 + b >= NBUF)
                    def _wait_prev_write():
                        pltpu.make_async_copy(
                            rows[b], out_hbm.at[pl.ds(0, CHUNK)], wsems[b]
                        ).wait()

                    cp = pltpu.make_async_copy(
                        w_v.at[idx_v.at[g * NBUF + b]], rows[b], gsems[b]
                    )
                    cp.start()
                    gathers.append(cp)
                for b in range(NBUF):
                    gathers[b].wait()
                    pltpu.make_async_copy(
                        rows[b],
                        out_hbm.at[pl.ds((grow + b) * CHUNK, CHUNK)],
                        wsems[b],
                    ).start()
                return carry

            lax.fori_loop(0, groups_per_stage, group_body, 0)
            @pl.when(st + 2 < n_stages)
            def _prefetch_next():
                idx_fetch(st + 2, slot)

        idx_fetch(0, 0)
        if n_stages > 1:
            idx_fetch(1, 1)

        def super_body(ss, carry):
            process_stage(2 * ss, 0)
            process_stage(2 * ss + 1, 1)
            return carry

        lax.fori_loop(0, n_stages // 2, super_body, 0)
        for b in range(NBUF):
            pltpu.make_async_copy(
                rows[b], out_hbm.at[pl.ds(0, CHUNK)], wsems[b]
            ).wait()

    return sc_embed


def kernel(x, weight):
    n, m = x.shape
    total = n * m
    x2 = x.astype(jnp.int32).reshape(total // CHUNK, CHUNK)
    out = _make_sc_embed(total // CHUNK)(x2, weight)
    return out.reshape(n, m, D)


# speedup vs baseline: 1.2769x; 1.0074x over previous
import functools

import jax
import jax.numpy as jnp
from jax import lax
from jax.experimental import pallas as pl
from jax.experimental.pallas import tpu as pltpu
from jax.experimental.pallas import tpu_sc as plsc

D = 128
NC = 2
NS = 16
NW = NC * NS
ST_R = 64
CA = 128
NSLOT = 2


@functools.lru_cache(maxsize=None)
def _make_sc_embed(n: int, m: int):
    cb = m - CA
    assert 0 < cb <= 128 and cb % 8 == 0 and m % 8 == 0
    rows_per_w = n // NW
    n_stages = rows_per_w // ST_R
    groups_per_stage = ST_R // NSLOT
    assert n == rows_per_w * NW
    assert rows_per_w == n_stages * ST_R and n_stages % 2 == 0
    assert ST_R == groups_per_stage * NSLOT

    mesh = plsc.VectorSubcoreMesh(core_axis_name="c", subcore_axis_name="s")

    @functools.partial(
        pl.kernel,
        mesh=mesh,
        out_type=jax.ShapeDtypeStruct((n * m, D), jnp.float32),
        scratch_types=(
            [pltpu.VMEM((ST_R, m), jnp.int32) for _ in range(2)]
            + [pltpu.VMEM_SHARED((5, D), jnp.float32)]
            + [pltpu.VMEM((CA, D), jnp.float32) for _ in range(NSLOT)]
            + [pltpu.VMEM((cb, D), jnp.float32) for _ in range(NSLOT)]
            + [pltpu.SemaphoreType.DMA for _ in range(4 * NSLOT + 2)]
        ),
    )
    def sc_embed(x_hbm, w_hbm, out_hbm, idx0, idx1, w_v, *bufs_and_sems):
        idx_bufs = (idx0, idx1)
        rows_a = bufs_and_sems[:NSLOT]
        rows_b = bufs_and_sems[NSLOT : 2 * NSLOT]
        sems = bufs_and_sems[2 * NSLOT :]
        gsems_a = sems[:NSLOT]
        gsems_b = sems[NSLOT : 2 * NSLOT]
        wsems_a = sems[2 * NSLOT : 3 * NSLOT]
        wsems_b = sems[3 * NSLOT : 4 * NSLOT]
        isems = sems[4 * NSLOT : 4 * NSLOT + 2]
        cid = lax.axis_index("c")
        sid = lax.axis_index("s")
        wid = sid * NC + cid
        xr0 = wid * rows_per_w

        @pl.when(sid == 0)
        def _stage_table():
            pltpu.sync_copy(w_hbm, w_v)

        plsc.subcore_barrier()

        def idx_fetch(st, slot):
            pltpu.make_async_copy(
                x_hbm.at[pl.ds(xr0 + st * ST_R, ST_R)],
                idx_bufs[slot],
                isems[slot],
            ).start()

        def process_stage(st, slot):
            idx_v = idx_bufs[slot]
            pltpu.make_async_copy(
                x_hbm.at[pl.ds(xr0, ST_R)], idx_v, isems[slot]
            ).wait()
            srow = xr0 + st * ST_R

            def group_body(g, carry):
                gathers = []
                for p in range(NSLOT):
                    lr = g * NSLOT + p

                    @pl.when(st * groups_per_stage + g >= 1)
                    def _wait_prev_writes():
                        pltpu.make_async_copy(
                            rows_a[p], out_hbm.at[pl.ds(0, CA)], wsems_a[p]
                        ).wait()
                        pltpu.make_async_copy(
                            rows_b[p], out_hbm.at[pl.ds(0, cb)], wsems_b[p]
                        ).wait()

                    ca = pltpu.make_async_copy(
                        w_v.at[idx_v.at[lr, pl.ds(0, CA)]], rows_a[p], gsems_a[p]
                    )
                    ca.start()
                    cbp = pltpu.make_async_copy(
                        w_v.at[idx_v.at[lr, pl.ds(CA, cb)]], rows_b[p], gsems_b[p]
                    )
                    cbp.start()
                    gathers.append((ca, cbp))
                for p in range(NSLOT):
                    lr = g * NSLOT + p
                    orow = (srow + lr) * m
                    ca, cbp = gathers[p]
                    ca.wait()
                    pltpu.make_async_copy(
                        rows_a[p], out_hbm.at[pl.ds(orow, CA)], wsems_a[p]
                    ).start()
                    cbp.wait()
                    pltpu.make_async_copy(
                        rows_b[p], out_hbm.at[pl.ds(orow + CA, cb)], wsems_b[p]
                    ).start()
                return carry

            lax.fori_loop(0, groups_per_stage, group_body, 0)
            @pl.when(st + 2 < n_stages)
            def _prefetch_next():
                idx_fetch(st + 2, slot)

        idx_fetch(0, 0)
        if n_stages > 1:
            idx_fetch(1, 1)

        def super_body(ss, carry):
            process_stage(2 * ss, 0)
            process_stage(2 * ss + 1, 1)
            return carry

        lax.fori_loop(0, n_stages // 2, super_body, 0)
        for p in range(NSLOT):
            pltpu.make_async_copy(
                rows_a[p], out_hbm.at[pl.ds(0, CA)], wsems_a[p]
            ).wait()
            pltpu.make_async_copy(
                rows_b[p], out_hbm.at[pl.ds(0, cb)], wsems_b[p]
            ).wait()

    return sc_embed


def kernel(x, weight):
    n, m = x.shape
    out = _make_sc_embed(n, m)(x.astype(jnp.int32), weight)
    return out.reshape(n, m, D)


# speedup vs baseline: 1.2791x; 1.0017x over previous
import functools

import jax
import jax.numpy as jnp
from jax import lax
from jax.experimental import pallas as pl
from jax.experimental.pallas import tpu as pltpu
from jax.experimental.pallas import tpu_sc as plsc

D = 128
NC = 2
NS = 16
NW = NC * NS
ST_R = 32
CA = 128
NSLOT = 4


@functools.lru_cache(maxsize=None)
def _make_sc_embed(n: int, m: int):
    cb = m - CA
    assert 0 < cb <= 128 and cb % 8 == 0 and m % 8 == 0
    rows_per_w = n // NW
    n_stages = rows_per_w // ST_R
    groups_per_stage = ST_R // NSLOT
    assert n == rows_per_w * NW
    assert rows_per_w == n_stages * ST_R and n_stages % 2 == 0
    assert ST_R == groups_per_stage * NSLOT

    mesh = plsc.VectorSubcoreMesh(core_axis_name="c", subcore_axis_name="s")

    @functools.partial(
        pl.kernel,
        mesh=mesh,
        out_type=jax.ShapeDtypeStruct((n * m, D), jnp.float32),
        scratch_types=(
            [pltpu.VMEM((ST_R, m), jnp.int32) for _ in range(2)]
            + [pltpu.VMEM_SHARED((5, D), jnp.float32)]
            + [pltpu.VMEM((CA, D), jnp.float32) for _ in range(NSLOT)]
            + [pltpu.VMEM((cb, D), jnp.float32) for _ in range(NSLOT)]
            + [pltpu.SemaphoreType.DMA for _ in range(4 * NSLOT + 2)]
        ),
    )
    def sc_embed(x_hbm, w_hbm, out_hbm, idx0, idx1, w_v, *bufs_and_sems):
        idx_bufs = (idx0, idx1)
        rows_a = bufs_and_sems[:NSLOT]
        rows_b = bufs_and_sems[NSLOT : 2 * NSLOT]
        sems = bufs_and_sems[2 * NSLOT :]
        gsems_a = sems[:NSLOT]
        gsems_b = sems[NSLOT : 2 * NSLOT]
        wsems_a = sems[2 * NSLOT : 3 * NSLOT]
        wsems_b = sems[3 * NSLOT : 4 * NSLOT]
        isems = sems[4 * NSLOT : 4 * NSLOT + 2]
        cid = lax.axis_index("c")
        sid = lax.axis_index("s")
        wid = sid * NC + cid
        xr0 = wid * rows_per_w

        @pl.when(sid == 0)
        def _stage_table():
            pltpu.sync_copy(w_hbm, w_v)

        plsc.subcore_barrier()

        def idx_fetch(st, slot):
            pltpu.make_async_copy(
                x_hbm.at[pl.ds(xr0 + st * ST_R, ST_R)],
                idx_bufs[slot],
                isems[slot],
            ).start()

        def process_stage(st, slot):
            idx_v = idx_bufs[slot]
            pltpu.make_async_copy(
                x_hbm.at[pl.ds(xr0, ST_R)], idx_v, isems[slot]
            ).wait()
            srow = xr0 + st * ST_R

            def group_body(g, carry):
                gathers = []
                for p in range(NSLOT):
                    lr = g * NSLOT + p

                    @pl.when(st * groups_per_stage + g >= 1)
                    def _wait_prev_writes():
                        pltpu.make_async_copy(
                            rows_a[p], out_hbm.at[pl.ds(0, CA)], wsems_a[p]
                        ).wait()
                        pltpu.make_async_copy(
                            rows_b[p], out_hbm.at[pl.ds(0, cb)], wsems_b[p]
                        ).wait()

                    ca = pltpu.make_async_copy(
                        w_v.at[idx_v.at[lr, pl.ds(0, CA)]], rows_a[p], gsems_a[p]
                    )
                    ca.start()
                    cbp = pltpu.make_async_copy(
                        w_v.at[idx_v.at[lr, pl.ds(CA, cb)]], rows_b[p], gsems_b[p]
                    )
                    cbp.start()
                    gathers.append((ca, cbp))
                for p in range(NSLOT):
                    lr = g * NSLOT + p
                    orow = (srow + lr) * m
                    ca, cbp = gathers[p]
                    ca.wait()
                    pltpu.make_async_copy(
                        rows_a[p], out_hbm.at[pl.ds(orow, CA)], wsems_a[p]
                    ).start()
                    cbp.wait()
                    pltpu.make_async_copy(
                        rows_b[p], out_hbm.at[pl.ds(orow + CA, cb)], wsems_b[p]
                    ).start()
                return carry

            lax.fori_loop(0, groups_per_stage, group_body, 0)
            @pl.when(st + 2 < n_stages)
            def _prefetch_next():
                idx_fetch(st + 2, slot)

        idx_fetch(0, 0)
        if n_stages > 1:
            idx_fetch(1, 1)

        def super_body(ss, carry):
            process_stage(2 * ss, 0)
            process_stage(2 * ss + 1, 1)
            return carry

        lax.fori_loop(0, n_stages // 2, super_body, 0)
        for p in range(NSLOT):
            pltpu.make_async_copy(
                rows_a[p], out_hbm.at[pl.ds(0, CA)], wsems_a[p]
            ).wait()
            pltpu.make_async_copy(
                rows_b[p], out_hbm.at[pl.ds(0, cb)], wsems_b[p]
            ).wait()

    return sc_embed


def kernel(x, weight):
    n, m = x.shape
    out = _make_sc_embed(n, m)(x.astype(jnp.int32), weight)
    return out.reshape(n, m, D)


# speedup vs baseline: 1.2797x; 1.0005x over previous
import functools

import jax
import jax.numpy as jnp
from jax import lax
from jax.experimental import pallas as pl
from jax.experimental.pallas import tpu as pltpu
from jax.experimental.pallas import tpu_sc as plsc

D = 128
NC = 2
NS = 16
NW = NC * NS
ST_R = 32
CA = 128
NSLOT = 4


@functools.lru_cache(maxsize=None)
def _make_sc_embed(n: int, m: int):
    cb = m - CA
    assert 0 < cb <= 128 and cb % 8 == 0 and m % 8 == 0
    rows_per_w = n // NW
    n_stages = rows_per_w // ST_R
    groups_per_stage = ST_R // NSLOT
    assert n == rows_per_w * NW
    assert rows_per_w == n_stages * ST_R and n_stages % 2 == 0
    assert ST_R == groups_per_stage * NSLOT

    mesh = plsc.VectorSubcoreMesh(core_axis_name="c", subcore_axis_name="s")

    @functools.partial(
        pl.kernel,
        mesh=mesh,
        out_type=jax.ShapeDtypeStruct((n * m, D), jnp.float32),
        scratch_types=(
            [pltpu.VMEM((ST_R, m), jnp.int32) for _ in range(2)]
            + [pltpu.VMEM_SHARED((5, D), jnp.float32)]
            + [pltpu.VMEM((m, D), jnp.float32) for _ in range(NSLOT)]
            + [pltpu.SemaphoreType.DMA for _ in range(2 * NSLOT + 2)]
        ),
    )
    def sc_embed(x_hbm, w_hbm, out_hbm, idx0, idx1, w_v, *bufs_and_sems):
        idx_bufs = (idx0, idx1)
        rows = bufs_and_sems[:NSLOT]
        gsems = bufs_and_sems[NSLOT : 2 * NSLOT]
        wsems = bufs_and_sems[2 * NSLOT : 3 * NSLOT]
        isems = bufs_and_sems[3 * NSLOT : 3 * NSLOT + 2]
        cid = lax.axis_index("c")
        sid = lax.axis_index("s")
        wid = sid * NC + cid
        xr0 = wid * rows_per_w

        @pl.when(sid == 0)
        def _stage_table():
            pltpu.sync_copy(w_hbm, w_v)

        plsc.subcore_barrier()

        def idx_fetch(st, slot):
            pltpu.make_async_copy(
                x_hbm.at[pl.ds(xr0 + st * ST_R, ST_R)],
                idx_bufs[slot],
                isems[slot],
            ).start()

        def process_stage(st, slot):
            idx_v = idx_bufs[slot]
            pltpu.make_async_copy(
                x_hbm.at[pl.ds(xr0, ST_R)], idx_v, isems[slot]
            ).wait()
            srow = xr0 + st * ST_R

            def group_body(g, carry):
                gathers = []
                for p in range(NSLOT):
                    lr = g * NSLOT + p

                    @pl.when(st * groups_per_stage + g >= 1)
                    def _wait_prev_write():
                        pltpu.make_async_copy(
                            rows[p], out_hbm.at[pl.ds(0, m)], wsems[p]
                        ).wait()

                    ca = pltpu.make_async_copy(
                        w_v.at[idx_v.at[lr, pl.ds(0, CA)]],
                        rows[p].at[pl.ds(0, CA)],
                        gsems[p],
                    )
                    ca.start()
                    cbp = pltpu.make_async_copy(
                        w_v.at[idx_v.at[lr, pl.ds(CA, cb)]],
                        rows[p].at[pl.ds(CA, cb)],
                        gsems[p],
                    )
                    cbp.start()
                    gathers.append((ca, cbp))
                for p in range(NSLOT):
                    lr = g * NSLOT + p
                    ca, cbp = gathers[p]
                    ca.wait()
                    cbp.wait()
                    pltpu.make_async_copy(
                        rows[p],
                        out_hbm.at[pl.ds((srow + lr) * m, m)],
                        wsems[p],
                    ).start()
                return carry

            lax.fori_loop(0, groups_per_stage, group_body, 0)
            @pl.when(st + 2 < n_stages)
            def _prefetch_next():
                idx_fetch(st + 2, slot)

        idx_fetch(0, 0)
        if n_stages > 1:
            idx_fetch(1, 1)

        def super_body(ss, carry):
            process_stage(2 * ss, 0)
            process_stage(2 * ss + 1, 1)
            return carry

        lax.fori_loop(0, n_stages // 2, super_body, 0)
        for p in range(NSLOT):
            pltpu.make_async_copy(
                rows[p], out_hbm.at[pl.ds(0, m)], wsems[p]
            ).wait()

    return sc_embed


def kernel(x, weight):
    n, m = x.shape
    out = _make_sc_embed(n, m)(x.astype(jnp.int32), weight)
    return out.reshape(n, m, D)


# speedup vs baseline: 1.2803x; 1.0004x over previous
import functools

import jax
import jax.numpy as jnp
from jax import lax
from jax.experimental import pallas as pl
from jax.experimental.pallas import tpu as pltpu
from jax.experimental.pallas import tpu_sc as plsc

NC = 2
NS = 16
NW = NC * NS
ST_R = 32
CA = 128
NSLOT = 4


@functools.lru_cache(maxsize=None)
def _make_sc_embed(n: int, m: int, v: int, d: int):
    cb = m - CA
    assert 0 < cb <= 128 and cb % 8 == 0 and m % 8 == 0
    rows_per_w = n // NW
    n_stages = rows_per_w // ST_R
    groups_per_stage = ST_R // NSLOT
    assert n == rows_per_w * NW
    assert rows_per_w == n_stages * ST_R and n_stages % 2 == 0
    assert ST_R == groups_per_stage * NSLOT

    mesh = plsc.VectorSubcoreMesh(core_axis_name="c", subcore_axis_name="s")

    @functools.partial(
        pl.kernel,
        mesh=mesh,
        out_type=jax.ShapeDtypeStruct((n * m, d), jnp.float32),
        scratch_types=(
            [pltpu.VMEM((ST_R, m), jnp.int32) for _ in range(2)]
            + [pltpu.VMEM_SHARED((v, d), jnp.float32)]
            + [pltpu.VMEM((m, d), jnp.float32) for _ in range(NSLOT)]
            + [pltpu.SemaphoreType.DMA for _ in range(2 * NSLOT + 2)]
        ),
    )
    def sc_embed(x_hbm, w_hbm, out_hbm, idx0, idx1, w_v, *bufs_and_sems):
        idx_bufs = (idx0, idx1)
        rows = bufs_and_sems[:NSLOT]
        gsems = bufs_and_sems[NSLOT : 2 * NSLOT]
        wsems = bufs_and_sems[2 * NSLOT : 3 * NSLOT]
        isems = bufs_and_sems[3 * NSLOT : 3 * NSLOT + 2]
        cid = lax.axis_index("c")
        sid = lax.axis_index("s")
        wid = sid * NC + cid
        xr0 = wid * rows_per_w

        @pl.when(sid == 0)
        def _stage_table():
            pltpu.sync_copy(w_hbm, w_v)

        plsc.subcore_barrier()

        def idx_fetch(st, slot):
            pltpu.make_async_copy(
                x_hbm.at[pl.ds(xr0 + st * ST_R, ST_R)],
                idx_bufs[slot],
                isems[slot],
            ).start()

        def process_stage(st, slot):
            idx_v = idx_bufs[slot]
            pltpu.make_async_copy(
                x_hbm.at[pl.ds(xr0, ST_R)], idx_v, isems[slot]
            ).wait()
            srow = xr0 + st * ST_R

            def group_body(g, carry):
                gathers = []
                for p in range(NSLOT):
                    lr = g * NSLOT + p

                    @pl.when(st * groups_per_stage + g >= 1)
                    def _wait_prev_write():
                        pltpu.make_async_copy(
                            rows[p], out_hbm.at[pl.ds(0, m)], wsems[p]
                        ).wait()

                    ca = pltpu.make_async_copy(
                        w_v.at[idx_v.at[lr, pl.ds(0, CA)]],
                        rows[p].at[pl.ds(0, CA)],
                        gsems[p],
                    )
                    ca.start()
                    cbp = pltpu.make_async_copy(
                        w_v.at[idx_v.at[lr, pl.ds(CA, cb)]],
                        rows[p].at[pl.ds(CA, cb)],
                        gsems[p],
                    )
                    cbp.start()
                    gathers.append((ca, cbp))
                for p in range(NSLOT):
                    lr = g * NSLOT + p
                    ca, cbp = gathers[p]
                    ca.wait()
                    cbp.wait()
                    pltpu.make_async_copy(
                        rows[p],
                        out_hbm.at[pl.ds((srow + lr) * m, m)],
                        wsems[p],
                    ).start()
                return carry

            lax.fori_loop(0, groups_per_stage, group_body, 0)
            @pl.when(st + 2 < n_stages)
            def _prefetch_next():
                idx_fetch(st + 2, slot)

        idx_fetch(0, 0)
        if n_stages > 1:
            idx_fetch(1, 1)

        def super_body(ss, carry):
            process_stage(2 * ss, 0)
            process_stage(2 * ss + 1, 1)
            return carry

        lax.fori_loop(0, n_stages // 2, super_body, 0)
        for p in range(NSLOT):
            pltpu.make_async_copy(
                rows[p], out_hbm.at[pl.ds(0, m)], wsems[p]
            ).wait()

    return sc_embed


def kernel(x, weight):
    n, m = x.shape
    v, d = weight.shape
    out = _make_sc_embed(n, m, v, d)(x.astype(jnp.int32), weight)
    return out.reshape(n, m, d)
